# baseline (speedup 1.0000x reference)
import jax
import jax.numpy as jnp
from jax import lax
from jax.experimental import pallas as pl

_DIM_2 = 128
_BB = 256
_WIN = 5504
_BIGF = 3.0e38


def _round_bf16(v):
    u = lax.bitcast_convert_type(v, jnp.uint32)
    r = (u + jnp.uint32(0x7FFF) + ((u >> 16) & jnp.uint32(1))) & jnp.uint32(0xFFFF0000)
    return lax.bitcast_convert_type(r, jnp.float32)


def _bmu_body(xm2_ref, w_ref, xsq_ref, wsq_ref, iota_ref, hilo_ref, out_ref):
    xm2 = xm2_ref[...]
    xsq = xsq_ref[...]
    k = w_ref.shape[0]
    bounds = list(range(0, k, _WIN)) + [k]

    cur_v = None
    cur_i = None
    for w in range(len(bounds) - 1):
        lo, hi = bounds[w], bounds[w + 1]
        mm = lax.dot_general(
            xm2, w_ref[lo:hi, :],
            dimension_numbers=(((1,), (1,)), ((), ())),
            preferred_element_type=jnp.float32,
        )
        d2 = (xsq + wsq_ref[:, lo:hi]) + mm
        m = jnp.min(d2, axis=1, keepdims=True)
        onehot = jnp.where(d2 == m, 1.0, 0.0)
        agg = lax.dot_general(
            onehot, hilo_ref[lo:hi, :],
            dimension_numbers=(((1,), (0,)), ((), ())),
            preferred_element_type=jnp.float32,
        )
        tie = jnp.max(agg[:, 2]) > 1.5

        def _exact_bi():
            return jnp.min(jnp.where(d2 == m, iota_ref[:, lo:hi], _BIGF),
                           axis=1, keepdims=True)

        def _sum_bi():
            return (agg[:, 0:1] * 128.0 + agg[:, 1:2])

        bi = lax.cond(tie, _exact_bi, _sum_bi)
        dw = jnp.sqrt(jnp.maximum(m, 0.0))
        if w == 0:
            cur_v, cur_i = _round_bf16(dw), bi
        else:
            take = dw < cur_v
            cur_v = jnp.where(take, _round_bf16(dw), cur_v)
            cur_i = jnp.where(take, bi, cur_i)

    idx = cur_i.astype(jnp.int32)
    out_ref[...] = jnp.concatenate([idx >> 7, idx & (_DIM_2 - 1)], axis=1)


@jax.jit
def kernel(x, weights):
    b, d = x.shape
    k, _ = weights.shape
    nbb = b // _BB

    x_sq = jnp.sum(x * x, axis=1, keepdims=True)
    w_sq = jnp.sum(weights * weights, axis=1)[None, :]
    xm2 = (-2.0 * x).astype(jnp.bfloat16)
    w16 = weights.astype(jnp.bfloat16)
    iota_k = jnp.arange(k, dtype=jnp.float32)[None, :]
    ar = jnp.arange(k, dtype=jnp.int32)
    hilo = jnp.stack(
        [(ar >> 7).astype(jnp.float32), (ar & 127).astype(jnp.float32),
         jnp.ones((k,), jnp.float32)], axis=1)

    return pl.pallas_call(
        _bmu_body,
        grid=(nbb,),
        in_specs=[
            pl.BlockSpec((_BB, d), lambda ib: (ib, 0)),
            pl.BlockSpec((k, d), lambda ib: (0, 0)),
            pl.BlockSpec((_BB, 1), lambda ib: (ib, 0)),
            pl.BlockSpec((1, k), lambda ib: (0, 0)),
            pl.BlockSpec((1, k), lambda ib: (0, 0)),
            pl.BlockSpec((k, 3), lambda ib: (0, 0)),
        ],
        out_specs=pl.BlockSpec((_BB, 2), lambda ib: (ib, 0)),
        out_shape=jax.ShapeDtypeStruct((b, 2), jnp.int32),
    )(xm2, w16, x_sq, w_sq, iota_k, hilo)


# speedup vs baseline: 2.0580x; 2.0580x over previous
import jax
import jax.numpy as jnp
from jax import lax
from jax.experimental import pallas as pl

_DIM_2 = 128
_BB = 256
_WIN = 5504
_BIGF = 3.0e38


def _round_bf16(v):
    u = lax.bitcast_convert_type(v, jnp.uint32)
    r = (u + jnp.uint32(0x7FFF) + ((u >> 16) & jnp.uint32(1))) & jnp.uint32(0xFFFF0000)
    return lax.bitcast_convert_type(r, jnp.float32)


def _bmu_body(x_ref, w_ref, wsq_ref, out_ref):
    x = x_ref[...]
    xm2 = (-2.0 * x).astype(jnp.bfloat16)
    xsq = jnp.sum(x * x, axis=1, keepdims=True)
    k = w_ref.shape[0]
    bounds = list(range(0, k, _WIN)) + [k]

    cur_v = None
    cur_i = None
    for w in range(len(bounds) - 1):
        lo, hi = bounds[w], bounds[w + 1]
        mm = lax.dot_general(
            xm2, w_ref[lo:hi, :],
            dimension_numbers=(((1,), (1,)), ((), ())),
            preferred_element_type=jnp.float32,
        )
        d2 = (xsq + wsq_ref[:, lo:hi]) + mm
        m = jnp.min(d2, axis=1, keepdims=True)
        iota = (lax.broadcasted_iota(jnp.int32, (1, hi - lo), 1)
                .astype(jnp.float32) + float(lo))
        bi = jnp.min(jnp.where(d2 == m, iota, _BIGF),
                     axis=1, keepdims=True)
        dw = jnp.sqrt(jnp.maximum(m, 0.0))
        if w == 0:
            cur_v, cur_i = _round_bf16(dw), bi
        else:
            take = dw < cur_v
            cur_v = jnp.where(take, _round_bf16(dw), cur_v)
            cur_i = jnp.where(take, bi, cur_i)

    idx = cur_i.astype(jnp.int32)
    out_ref[...] = jnp.concatenate([idx >> 7, idx & (_DIM_2 - 1)], axis=1)


@jax.jit
def kernel(x, weights):
    b, d = x.shape
    k, _ = weights.shape
    nbb = b // _BB

    w_sq = jnp.sum(weights * weights, axis=1)[None, :]
    w16 = weights.astype(jnp.bfloat16)

    return pl.pallas_call(
        _bmu_body,
        grid=(nbb,),
        in_specs=[
            pl.BlockSpec((_BB, d), lambda ib: (ib, 0)),
            pl.BlockSpec((k, d), lambda ib: (0, 0)),
            pl.BlockSpec((1, k), lambda ib: (0, 0)),
        ],
        out_specs=pl.BlockSpec((_BB, 2), lambda ib: (ib, 0)),
        out_shape=jax.ShapeDtypeStruct((b, 2), jnp.int32),
    )(x, w16, w_sq)


# speedup vs baseline: 2.1673x; 1.0531x over previous
import jax
import jax.numpy as jnp
from jax import lax
from jax.experimental import pallas as pl

_DIM_2 = 128
_BB = 512
_WIN = 5504
_BIGF = 3.0e38


def _round_bf16(v):
    u = lax.bitcast_convert_type(v, jnp.uint32)
    r = (u + jnp.uint32(0x7FFF) + ((u >> 16) & jnp.uint32(1))) & jnp.uint32(0xFFFF0000)
    return lax.bitcast_convert_type(r, jnp.float32)


def _bmu_body(x_ref, w_ref, wsq_ref, out_ref):
    x = x_ref[...]
    xm2 = (-2.0 * x).astype(jnp.bfloat16)
    xsq = jnp.sum(x * x, axis=1, keepdims=True)
    k = w_ref.shape[0]
    bounds = list(range(0, k, _WIN)) + [k]

    cur_v = None
    cur_i = None
    for w in range(len(bounds) - 1):
        lo, hi = bounds[w], bounds[w + 1]
        mm = lax.dot_general(
            xm2, w_ref[lo:hi, :],
            dimension_numbers=(((1,), (1,)), ((), ())),
            preferred_element_type=jnp.float32,
        )
        d2 = (xsq + wsq_ref[:, lo:hi]) + mm
        m = jnp.min(d2, axis=1, keepdims=True)
        iota = (lax.broadcasted_iota(jnp.int32, (1, hi - lo), 1)
                .astype(jnp.float32) + float(lo))
        bi = jnp.min(jnp.where(d2 == m, iota, _BIGF),
                     axis=1, keepdims=True)
        dw = jnp.sqrt(jnp.maximum(m, 0.0))
        if w == 0:
            cur_v, cur_i = _round_bf16(dw), bi
        else:
            take = dw < cur_v
            cur_v = jnp.where(take, _round_bf16(dw), cur_v)
            cur_i = jnp.where(take, bi, cur_i)

    idx = cur_i.astype(jnp.int32)
    out_ref[...] = jnp.concatenate([idx >> 7, idx & (_DIM_2 - 1)], axis=1)


@jax.jit
def kernel(x, weights):
    b, d = x.shape
    k, _ = weights.shape
    nbb = b // _BB

    w_sq = jnp.sum(weights * weights, axis=1)[None, :]
    w16 = weights.astype(jnp.bfloat16)

    return pl.pallas_call(
        _bmu_body,
        grid=(nbb,),
        in_specs=[
            pl.BlockSpec((_BB, d), lambda ib: (ib, 0)),
            pl.BlockSpec((k, d), lambda ib: (0, 0)),
            pl.BlockSpec((1, k), lambda ib: (0, 0)),
        ],
        out_specs=pl.BlockSpec((_BB, 2), lambda ib: (ib, 0)),
        out_shape=jax.ShapeDtypeStruct((b, 2), jnp.int32),
    )(x, w16, w_sq)
